# Initial kernel scaffold; baseline (speedup 1.0000x reference)
import jax
import jax.numpy as jnp
from jax.experimental import pallas as pl


def kernel(input, pat):
    raise NotImplementedError("write your pallas kernel here")


# speedup vs baseline: 2.5894x; 2.5894x over previous
import jax
import jax.numpy as jnp
from jax import lax
from jax.experimental import pallas as pl
from jax.experimental.pallas import tpu as pltpu

DECAY = 0.999
UPDATE_RATE = 1.0
THRESH = 0.9
TEMP = 10.0
EPS = 1e-10


def _rnn_kernel(inp_ref, pat_ref, out_ref, pats_ref, pat_scratch):
    t = pl.program_id(1)

    @pl.when(t == 0)
    def _():
        pat_scratch[...] = pat_ref[...]

    Bc, P, H = pat_scratch.shape
    pat = pat_scratch[...]
    h = inp_ref[0]
    pat2d = pat.reshape(Bc * P, H)

    raw_all = lax.dot_general(
        h, pat2d, (((1,), (1,)), ((), ())),
        preferred_element_type=jnp.float32)
    raw = jnp.concatenate(
        [raw_all[b:b + 1, b * P:(b + 1) * P] for b in range(Bc)], axis=0)

    mx = jnp.max(raw, axis=1, keepdims=True)
    masked = jnp.where(raw >= THRESH * mx, raw, 0.0)
    z = masked / mx * TEMP
    z = z - jnp.max(z, axis=1, keepdims=True)
    e = jnp.exp(z)
    resp = e / jnp.sum(e, axis=1, keepdims=True)

    row = lax.broadcasted_iota(jnp.int32, (Bc, P), 0)
    resp_blk = jnp.concatenate(
        [jnp.where(row == b, resp, 0.0) for b in range(Bc)], axis=1)
    new_h = lax.dot_general(
        resp_blk, pat2d, (((1,), (0,)), ((), ())),
        preferred_element_type=jnp.float32)

    up = resp[:, :, None] * h[:, None, :]
    newp = DECAY * pat + UPDATE_RATE * up
    nrm = jnp.sqrt(jnp.sum(newp * newp, axis=2, keepdims=True))
    new_pat = newp / (nrm + EPS)

    pat_scratch[...] = new_pat
    out_ref[0] = new_h
    pats_ref[0] = new_pat


def kernel(input, pat):
    T, B, H = input.shape
    _, P, _ = pat.shape
    BC = 2
    Bc = B // BC

    out, all_pats = pl.pallas_call(
        _rnn_kernel,
        grid=(BC, T),
        in_specs=[
            pl.BlockSpec((1, Bc, H), lambda i, t: (t, i, 0)),
            pl.BlockSpec((Bc, P, H), lambda i, t: (i, 0, 0)),
        ],
        out_specs=[
            pl.BlockSpec((1, Bc, H), lambda i, t: (t, i, 0)),
            pl.BlockSpec((1, Bc, P, H), lambda i, t: (t, i, 0, 0)),
        ],
        out_shape=[
            jax.ShapeDtypeStruct((T, B, H), input.dtype),
            jax.ShapeDtypeStruct((T, B, P, H), input.dtype),
        ],
        scratch_shapes=[pltpu.VMEM((Bc, P, H), jnp.float32)],
        compiler_params=pltpu.CompilerParams(
            dimension_semantics=("parallel", "arbitrary"),
        ),
        name="pat_softmax_rnn",
    )(input, pat)
    return out, all_pats
